# baseline (speedup 1.0000x reference)
import functools

import jax
import jax.numpy as jnp
from jax import lax
from jax.experimental import pallas as pl
from jax.experimental.pallas import tpu as pltpu
from jax.experimental.pallas import tpu_sc as plsc

F = 64
NC = 2
NS = 16
VR = 16
NBUF = 3


def _spmm_body(fc, sb, bb, n_dst_pad, g_cap, x_hbm, sidx_hbm, didx_hbm,
               val_hbm, cnt_hbm, out_hbm, sidx_v, didx_v, val_v, cnt_v,
               rows_v, acc, ssem, gsem, csem):
    c = lax.axis_index("c")
    s = lax.axis_index("s")
    w = c * NS + s
    zrows = n_dst_pad // NS
    KG = sb // bb
    P = F // fc
    HC = fc // 32

    pltpu.sync_copy(cnt_hbm.at[w], cnt_v)
    gr = cnt_v[...][0]

    for p in range(P):
        def _zb(i, carry):
            for j in range(fc // VR):
                rows_v[0, i, pl.ds(j * VR, VR)] = jnp.zeros((VR,), jnp.float32)
            return carry

        lax.fori_loop(0, bb, _zb, 0)

        def _zero(i, carry):
            pltpu.sync_copy(rows_v.at[0, pl.ds(0, bb)],
                            acc.at[pl.ds(s * zrows + i * bb, bb)])
            return carry

        lax.fori_loop(0, zrows // bb, _zero, 0)
        plsc.subcore_barrier()

        def _stage(g, b):
            row = w * g_cap + g
            pltpu.async_copy(sidx_hbm.at[pl.ds(row * sb, sb)],
                             sidx_v.at[b], ssem.at[b])
            pltpu.async_copy(didx_hbm.at[pl.ds(row * KG, KG)],
                             didx_v.at[b], ssem.at[b])
            pltpu.async_copy(val_hbm.at[pl.ds(row * sb, sb)],
                             val_v.at[b], ssem.at[b])

        def _stage_wait(b):
            pltpu.make_async_copy(sidx_hbm.at[pl.ds(0, sb)],
                                  sidx_v.at[b], ssem.at[b]).wait()
            pltpu.make_async_copy(didx_hbm.at[pl.ds(0, KG)],
                                  didx_v.at[b], ssem.at[b]).wait()
            pltpu.make_async_copy(val_hbm.at[pl.ds(0, sb)],
                                  val_v.at[b], ssem.at[b]).wait()

        def _gather(b):
            for k in range(KG):
                pltpu.async_copy(
                    x_hbm.at[p].at[sidx_v.at[b, pl.ds(k * bb, bb)]],
                    rows_v.at[b, pl.ds(k * bb, bb)], gsem.at[b])

        def _gather_wait(b):
            for k in range(KG):
                pltpu.make_async_copy(
                    x_hbm.at[p].at[sidx_v.at[b, pl.ds(k * bb, bb)]],
                    rows_v.at[b, pl.ds(k * bb, bb)], gsem.at[b]).wait()

        def _scatter(b):
            for k in range(KG):
                pltpu.async_copy(rows_v.at[b, pl.ds(k * bb, bb)],
                                 acc.at[didx_v.at[b, k]], csem.at[b],
                                 add=True)

        def _scatter_wait(b):
            for k in range(KG):
                pltpu.make_async_copy(rows_v.at[b, pl.ds(k * bb, bb)],
                                      acc.at[didx_v.at[b, k]],
                                      csem.at[b]).wait()

        @pl.when(gr > 0)
        def _():
            _stage(0, 0)
            _stage_wait(0)
            _gather(0)

        @pl.when(gr > 1)
        def _():
            _stage(1, 1)

        def _step(g, carry):
            @pl.when(g < gr)
            def _():
                _step_body(g)
            return carry

        def _step_body(g):
            b = lax.rem(g, NBUF)
            bn = lax.rem(g + 1, NBUF)
            bs = lax.rem(g + 2, NBUF)

            @pl.when(g + 1 < gr)
            def _():
                _stage_wait(bn)

            @pl.when(g >= 1)
            def _():
                _scatter_wait(bs)

            @pl.when(g + 2 < gr)
            def _():
                _stage(g + 2, bs)

            _gather_wait(b)

            @pl.when(g + 1 < gr)
            def _():
                _gather(bn)

            def _scale_grp(gg, inner):
                vv = val_v[b, pl.ds(gg * VR, VR)]
                for j in range(VR):
                    bc = vv.at[jnp.full((32,), j, jnp.int32)].get(
                        mode='promise_in_bounds')
                    r = gg * VR + j
                    for h in range(HC):
                        rows_v[b, r, pl.ds(h * 32, 32)] = (
                            rows_v[b, r, pl.ds(h * 32, 32)] * bc)
                return inner

            lax.fori_loop(0, sb // VR, _scale_grp, 0)
            _scatter(b)

        lax.fori_loop(0, g_cap, _step, 0)

        @pl.when(gr > 0)
        def _():
            _scatter_wait(lax.rem(gr - 1, NBUF))

        plsc.subcore_barrier()

        pltpu.sync_copy(acc.at[pl.ds(s * zrows, zrows)],
                        out_hbm.at[p, c, pl.ds(s * zrows, zrows)])
        plsc.subcore_barrier()


def _make_spmm(fc, sb, bb, n_dst_pad, g_cap):
    mesh = plsc.VectorSubcoreMesh(core_axis_name="c", subcore_axis_name="s",
                                  num_cores=NC, num_subcores=NS)
    kg = sb // bb
    return pl.kernel(
        functools.partial(_spmm_body, fc, sb, bb, n_dst_pad, g_cap),
        out_type=jax.ShapeDtypeStruct((F // fc, NC, n_dst_pad, fc),
                                      jnp.float32),
        mesh=mesh,
        compiler_params=pltpu.CompilerParams(use_tc_tiling_on_sc=False),
        scratch_types=[
            pltpu.VMEM((NBUF, sb), jnp.int32),
            pltpu.VMEM((NBUF, kg, bb), jnp.int32),
            pltpu.VMEM((NBUF, sb), jnp.float32),
            pltpu.VMEM((VR,), jnp.int32),
            pltpu.VMEM((NBUF, sb, fc), jnp.float32),
            pltpu.VMEM_SHARED((n_dst_pad, fc), jnp.float32),
            pltpu.SemaphoreType.DMA((NBUF,)),
            pltpu.SemaphoreType.DMA((NBUF,)),
            pltpu.SemaphoreType.DMA((NBUF,)),
        ],
    )


def _ceil_to(x, m):
    return (x + m - 1) // m * m


def _split(x, fc):
    n = x.shape[0]
    return x.reshape(n, F // fc, fc).transpose(1, 0, 2)


SB_U = 64
SB_I = 256
BB_I = 128


def kernel(embed_user, embed_item, u_idx, i_idx, ui_val, iu_val, adj_val, d_i, d_j):
    n_users, _ = embed_user.shape
    n_items = embed_item.shape[0]
    n_edges = u_idx.shape[0]

    def prep_split(n_dst, sb, bb, src, dst, vals):
        s0 = (n_dst + 1) // 2
        g_cap = -(-n_edges // (NS * sb))
        cap_tile = g_cap * sb
        cap_sc = NS * cap_tile
        tot = NC * cap_sc

        src = src.astype(jnp.int32)
        dst = dst.astype(jnp.int32)
        half = (dst >= s0).astype(jnp.int32)
        dstl = dst - half * s0
        c1 = jnp.cumsum(half)
        n1 = c1[-1]
        n0 = n_edges - n1
        rank1 = c1 - 1
        rank0 = (jnp.arange(n_edges, dtype=jnp.int32) - c1)
        rank = jnp.where(half == 1, rank1, rank0)
        chunk0 = jnp.maximum((n0 + NS - 1) // NS, 1)
        chunk1 = jnp.maximum((n1 + NS - 1) // NS, 1)
        chunk = jnp.where(half == 1, chunk1, chunk0)
        t = rank // chunk
        off = rank - t * chunk
        pos = half * cap_sc + t * cap_tile + off

        ts = jnp.arange(NS, dtype=jnp.int32)
        e0 = jnp.clip(n0 - ts * chunk0, 0, chunk0)
        e1 = jnp.clip(n1 - ts * chunk1, 0, chunk1)
        g0 = (e0 + sb - 1) // sb
        g1 = (e1 + sb - 1) // sb
        cnt = jnp.broadcast_to(
            jnp.concatenate([g0, g1])[:, None], (NC * NS, VR)
        ).astype(jnp.int32)

        src_a = jnp.zeros((tot,), jnp.int32).at[pos].set(src)
        dst_a = jnp.zeros((tot,), jnp.int32).at[pos].set(dstl)
        dst_a = dst_a.reshape(-1, bb)
        val_a = [jnp.zeros((tot,), jnp.float32).at[pos].set(v) for v in vals]
        return g_cap, s0, src_a, dst_a, val_a, cnt

    gu, s0_u, src_u, dst_u, (vu_adj, vu_ui), cnt_u = prep_split(
        n_users, SB_U, SB_U, i_idx, u_idx, [adj_val, ui_val])
    gi, s0_i, src_i, dst_i, (vi_adj, vi_iu), cnt_i = prep_split(
        n_items, SB_I, BB_I, u_idx, i_idx, [adj_val, iu_val])

    up = _ceil_to((n_users + 1) // 2, NS * SB_U)
    ip = _ceil_to((n_items + 1) // 2, NS * BB_I)

    spmm_u = _make_spmm(64, SB_U, SB_U, up, gu)
    spmm_i = _make_spmm(32, SB_I, BB_I, ip, gi)

    def run_u(x, v):
        o = spmm_u(x[None], src_u, dst_u, v, cnt_u)
        return jnp.concatenate(
            [o[0, 0, :s0_u], o[0, 1, :n_users - s0_u]], axis=0)

    def run_i(x, v):
        o = spmm_i(_split(x, 32), src_i, dst_i, v, cnt_i)
        full = jnp.concatenate([o[:, 0, :s0_i], o[:, 1, :n_items - s0_i]],
                               axis=1)
        return full.transpose(1, 0, 2).reshape(n_items, F)

    users_e = run_u(embed_item, vu_adj) + embed_user
    items_e = run_i(embed_user, vi_adj) + embed_item

    di = d_i[:, None]
    dj = d_j[:, None]

    g1u = run_u(items_e, vu_ui) + users_e * di
    g1i = run_i(users_e, vi_iu) + items_e * dj
    g2u = run_u(g1i, vu_ui) + g1u * di
    g2i = run_i(g1u, vi_iu) + g1i * dj
    g3u = run_u(g2i, vu_ui) + g2u * di
    g3i = run_i(g2u, vi_iu) + g2i * dj
    g4u = run_u(g3i, vu_ui) + g3u * di
    g4i = run_i(g3u, vi_iu) + g3i * dj

    gcn_users = users_e + g1u * (1 / 2) + g2u * (1 / 3) + g3u * (1 / 4) + g4u
    gcn_items = items_e + g1i * (1 / 2) + g2i * (1 / 3) + g3i * (1 / 4) + g4i
    return (gcn_users, gcn_items)


# speedup vs baseline: 3.0036x; 3.0036x over previous
import functools

import jax
import jax.numpy as jnp
from jax import lax
from jax.experimental import pallas as pl
from jax.experimental.pallas import tpu as pltpu
from jax.experimental.pallas import tpu_sc as plsc

F = 64
NC = 2
NS = 16
B = 128
VR = 16


NBUF = 3


def _spmm_body(fc, sb, n_dst_pad, g_per_tile, x_hbm, sidx_hbm, didx_hbm,
               val_hbm, out_hbm, sidx_v, didx_v, val_v, rows_v, zbuf_v, acc,
               ssem, gsem, csem):
    c = lax.axis_index("c")
    s = lax.axis_index("s")
    w = c * NS + s
    zrows = n_dst_pad // NS
    G = g_per_tile
    KG = sb // B
    P = F // fc
    H = fc // VR

    def _zb(r, carry):
        zbuf_v[r] = jnp.zeros((fc,), jnp.float32)
        return carry

    lax.fori_loop(0, B, _zb, 0)

    for p in range(P):
        def _zero(i, carry):
            pltpu.sync_copy(zbuf_v, acc.at[pl.ds(s * zrows + i * B, B)])
            return carry

        lax.fori_loop(0, zrows // B, _zero, 0)
        plsc.subcore_barrier()

        def _stage(g, b):
            row = w * G + g
            pltpu.async_copy(sidx_hbm.at[pl.ds(row * sb, sb)],
                             sidx_v.at[b], ssem.at[b])
            pltpu.async_copy(didx_hbm.at[pl.ds(row * KG, KG)],
                             didx_v.at[b], ssem.at[b])
            pltpu.async_copy(val_hbm.at[pl.ds(row * sb, sb)],
                             val_v.at[b], ssem.at[b])

        def _stage_wait(b):
            pltpu.make_async_copy(sidx_hbm.at[pl.ds(0, sb)],
                                  sidx_v.at[b], ssem.at[b]).wait()
            pltpu.make_async_copy(didx_hbm.at[pl.ds(0, KG)],
                                  didx_v.at[b], ssem.at[b]).wait()
            pltpu.make_async_copy(val_hbm.at[pl.ds(0, sb)],
                                  val_v.at[b], ssem.at[b]).wait()

        def _gather(b):
            for k in range(KG):
                pltpu.async_copy(
                    x_hbm.at[p].at[sidx_v.at[b, pl.ds(k * B, B)]],
                    rows_v.at[b, pl.ds(k * B, B)], gsem.at[b])

        def _gather_wait(b):
            for k in range(KG):
                pltpu.make_async_copy(
                    x_hbm.at[p].at[sidx_v.at[b, pl.ds(k * B, B)]],
                    rows_v.at[b, pl.ds(k * B, B)], gsem.at[b]).wait()

        def _scatter(b):
            for k in range(KG):
                pltpu.async_copy(rows_v.at[b, pl.ds(k * B, B)],
                                 acc.at[didx_v.at[b, k]], csem.at[b],
                                 add=True)

        def _scatter_wait(b):
            for k in range(KG):
                pltpu.make_async_copy(rows_v.at[b, pl.ds(k * B, B)],
                                      acc.at[didx_v.at[b, k]],
                                      csem.at[b]).wait()

        _stage(0, 0)
        _stage_wait(0)
        _gather(0)

        if G > 1:
            _stage(1, 1)

        def _step(g, carry):
            b = lax.rem(g, NBUF)
            bn = lax.rem(g + 1, NBUF)
            bs = lax.rem(g + 2, NBUF)

            @pl.when(g + 1 < G)
            def _():
                _stage_wait(bn)

            @pl.when(g >= 1)
            def _():
                _scatter_wait(bs)

            @pl.when(g + 2 < G)
            def _():
                _stage(g + 2, bs)

            _gather_wait(b)

            @pl.when(g + 1 < G)
            def _():
                _gather(bn)

            def _scale_grp(gg, inner):
                vv = val_v[b, pl.ds(gg * VR, VR)]
                for j in range(VR):
                    bc = vv.at[jnp.full((fc,), j, jnp.int32)].get(
                        mode='promise_in_bounds')
                    r = gg * VR + j
                    rows_v[b, r] = rows_v[b, r] * bc
                return inner

            lax.fori_loop(0, sb // VR, _scale_grp, 0)
            _scatter(b)
            return carry

        lax.fori_loop(0, G, _step, 0)
        _scatter_wait((G - 1) % NBUF)
        plsc.subcore_barrier()

        pltpu.sync_copy(acc.at[pl.ds(s * zrows, zrows)],
                        out_hbm.at[p, c, pl.ds(s * zrows, zrows)])
        plsc.subcore_barrier()


def _make_spmm(fc, sb, n_dst_pad, g_per_tile):
    mesh = plsc.VectorSubcoreMesh(core_axis_name="c", subcore_axis_name="s",
                                  num_cores=NC, num_subcores=NS)
    kg = sb // B
    return pl.kernel(
        functools.partial(_spmm_body, fc, sb, n_dst_pad, g_per_tile),
        out_type=jax.ShapeDtypeStruct((F // fc, NC, n_dst_pad, fc),
                                      jnp.float32),
        mesh=mesh,
        compiler_params=pltpu.CompilerParams(use_tc_tiling_on_sc=False),
        scratch_types=[
            pltpu.VMEM((NBUF, sb), jnp.int32),
            pltpu.VMEM((NBUF, kg, B), jnp.int32),
            pltpu.VMEM((NBUF, sb), jnp.float32),
            pltpu.VMEM((NBUF, sb, fc), jnp.float32),
            pltpu.VMEM((B, fc), jnp.float32),
            pltpu.VMEM_SHARED((n_dst_pad, fc), jnp.float32),
            pltpu.SemaphoreType.DMA((NBUF,)),
            pltpu.SemaphoreType.DMA((NBUF,)),
            pltpu.SemaphoreType.DMA((NBUF,)),
        ],
    )


def _ceil_to(x, m):
    return (x + m - 1) // m * m


def _split(x, fc):
    n = x.shape[0]
    return x.reshape(n, F // fc, fc).transpose(1, 0, 2)


def _to32(x4):
    n = x4.shape[1]
    return x4.reshape(2, 2, n, 16).transpose(0, 2, 1, 3).reshape(2, n, 32)


def _to16(x2):
    n = x2.shape[1]
    return x2.reshape(2, n, 2, 16).transpose(0, 2, 1, 3).reshape(4, n, 16)


def _unsplit16(x4):
    n = x4.shape[1]
    return x4.transpose(1, 0, 2).reshape(n, F)


SB_U = 128
SB_I = 512


def kernel(embed_user, embed_item, u_idx, i_idx, ui_val, iu_val, adj_val, d_i, d_j):
    n_users, _ = embed_user.shape
    n_items = embed_item.shape[0]
    n_edges = u_idx.shape[0]

    up = _ceil_to(n_users, NS * B)
    ip = _ceil_to(n_items, NS * B)

    def prep(sb, src, dst, vals):
        e_pad = _ceil_to(n_edges, NC * NS * sb)
        g_per_tile = e_pad // (NC * NS * sb)
        pad = e_pad - n_edges
        src_p = jnp.pad(src.astype(jnp.int32), (0, pad))
        dst_p = jnp.pad(dst.astype(jnp.int32), (0, pad)).reshape(-1, B)
        vals_p = [jnp.pad(v, (0, pad)) for v in vals]
        return g_per_tile, src_p, dst_p, vals_p

    gu, src_u, dst_u, (vu_ui, vu_adj) = prep(
        SB_U, i_idx, u_idx, [ui_val, adj_val])
    gi, src_i, dst_i, (vi_iu, vi_adj) = prep(
        SB_I, u_idx, i_idx, [iu_val, adj_val])

    spmm_u = _make_spmm(32, SB_U, up, gu)
    spmm_i = _make_spmm(16, SB_I, ip, gi)

    def run_u(x2, v2):
        o = spmm_u(x2, src_u, dst_u, v2)
        return (o[:, 0] + o[:, 1])[:, :n_users]

    def run_i(x4, v2):
        o = spmm_i(x4, src_i, dst_i, v2)
        return (o[:, 0] + o[:, 1])[:, :n_items]

    eu4 = _split(embed_user, 16)
    ei2 = _split(embed_item, 32)

    users_e = _to16(run_u(ei2, vu_adj)) + eu4
    items_e = run_i(eu4, vi_adj) + _to16(ei2)

    di = d_i[None, :, None]
    dj = d_j[None, :, None]

    def layer(prev_u4, prev_i4, term_u4, term_i4):
        gu_ = _to16(run_u(_to32(prev_i4), vu_ui)) + term_u4 * di
        gi_ = run_i(prev_u4, vi_iu) + term_i4 * dj
        return gu_, gi_

    g1u, g1i = layer(users_e, items_e, users_e, items_e)
    g2u, g2i = layer(g1u, g1i, g1u, g1i)
    g3u, g3i = layer(g2u, g2i, g2u, g2i)
    g4u, g4i = layer(g3u, g3i, g3u, g3i)

    gcn_users = users_e + g1u * (1 / 2) + g2u * (1 / 3) + g3u * (1 / 4) + g4u
    gcn_items = items_e + g1i * (1 / 2) + g2i * (1 / 3) + g3i * (1 / 4) + g4i
    return (_unsplit16(gcn_users), _unsplit16(gcn_items))
